# Initial kernel scaffold; baseline (speedup 1.0000x reference)
import jax
import jax.numpy as jnp
from jax.experimental import pallas as pl


def kernel(edge_index, emb, W1, b1, W2, b2):
    raise NotImplementedError("write your pallas kernel here")


# speedup vs baseline: 6.9630x; 6.9630x over previous
import functools

import jax
import jax.numpy as jnp
from jax import lax
from jax.experimental import pallas as pl
from jax.experimental.pallas import tpu as pltpu
from jax.experimental.pallas import tpu_sc as plsc

N = 10000
E = 320000
D = 128
NC = 2
NS = 16
NW = NC * NS
C = 128
NCH = 80
E_PAD = NW * NCH * C
NPAD = 10240
IPT = NPAD // NS
RPA = 624
TAIL0 = RPA * NS
TAILN = N - TAIL0

_mesh = plsc.VectorSubcoreMesh(core_axis_name="c", subcore_axis_name="s")


def _init_sliced(src, dst, s):
    pltpu.sync_copy(src.at[pl.ds(s * IPT, IPT)], dst.at[pl.ds(s * IPT, IPT)])


def _drain_sliced(src, dst, s):
    base = s * RPA
    pltpu.sync_copy(src.at[pl.ds(base, RPA)], dst.at[pl.ds(base, RPA)])

    @pl.when(s == NS - 1)
    def _():
        pltpu.sync_copy(src.at[pl.ds(TAIL0, TAILN)],
                        dst.at[pl.ds(TAIL0, TAILN)])


def _deg_body(colr, ones_h, zero_h, cnt_out, acc, colj, onesb):
    c = lax.axis_index("c")
    s = lax.axis_index("s")
    wid = s * NC + c
    _init_sliced(zero_h, acc, s)
    pltpu.sync_copy(ones_h, onesb)
    plsc.subcore_barrier()

    def body(j, carry):
        pltpu.sync_copy(colr.at[wid, j], colj)
        pltpu.sync_copy(onesb, acc.at[colj], add=True)
        return carry

    lax.fori_loop(0, NCH, body, 0)
    plsc.subcore_barrier()
    _drain_sliced(acc, cnt_out.at[c], s)


_deg_call = pl.kernel(
    _deg_body,
    out_type=jax.ShapeDtypeStruct((NC, N, D), jnp.float32),
    mesh=_mesh,
    scratch_types=[
        pltpu.VMEM_SHARED((NPAD, D), jnp.float32),
        pltpu.VMEM((C,), jnp.int32),
        pltpu.VMEM((C, D), jnp.float32),
    ],
)


def _agg_body(rowr, colr, g_h, zero_h, part_out, acc, rowj, colj, rowsv):
    c = lax.axis_index("c")
    s = lax.axis_index("s")
    wid = s * NC + c
    _init_sliced(zero_h, acc, s)
    plsc.subcore_barrier()

    def body(j, carry):
        pltpu.sync_copy(rowr.at[wid, j], rowj)
        pltpu.sync_copy(colr.at[wid, j], colj)
        pltpu.sync_copy(g_h.at[rowj], rowsv)
        pltpu.sync_copy(rowsv, acc.at[colj], add=True)
        return carry

    lax.fori_loop(0, NCH, body, 0)
    plsc.subcore_barrier()
    _drain_sliced(acc, part_out.at[c], s)


_agg_call = pl.kernel(
    _agg_body,
    out_type=jax.ShapeDtypeStruct((NC, N, D), jnp.float32),
    mesh=_mesh,
    scratch_types=[
        pltpu.VMEM_SHARED((NPAD, D), jnp.float32),
        pltpu.VMEM((C,), jnp.int32),
        pltpu.VMEM((C,), jnp.int32),
        pltpu.VMEM((C, D), jnp.float32),
    ],
)

BN = 1000


def _tc1_body(cnt_ref, x_ref, w_ref, g_ref, dis_ref):
    deg = cnt_ref[0] + cnt_ref[1] + 1.0
    dis = lax.rsqrt(deg)
    dis_ref[...] = dis[:, 0:16]
    h = jnp.dot(x_ref[...], w_ref[...], preferred_element_type=jnp.float32)
    g_ref[...] = h * dis[:, 0:1]


def _tc2_body(p_ref, g1_ref, dis_ref, b1_ref, w2_ref, g2_ref):
    dis = dis_ref[...][:, 0:1]
    x1 = jnp.maximum(dis * (p_ref[0] + p_ref[1] + g1_ref[...]) + b1_ref[...],
                     0.0)
    h = jnp.dot(x1, w2_ref[...], preferred_element_type=jnp.float32)
    g2_ref[...] = h * dis


def _tc3_body(q_ref, g2_ref, dis_ref, b2_ref, o_ref):
    dis = dis_ref[...][:, 0:1]
    o_ref[...] = dis * (q_ref[0] + q_ref[1] + g2_ref[...]) + b2_ref[...]


def _row_spec(w):
    return pl.BlockSpec((BN, w), lambda i: (i, 0))


def _part_spec(w):
    return pl.BlockSpec((NC, BN, w), lambda i: (0, i, 0))


def _full_spec(r, c):
    return pl.BlockSpec((r, c), lambda i: (0, 0))


_tc1 = pl.pallas_call(
    _tc1_body,
    grid=(N // BN,),
    in_specs=[_part_spec(D), _row_spec(D), _full_spec(D, D)],
    out_specs=[_row_spec(D), _row_spec(16)],
    out_shape=[jax.ShapeDtypeStruct((N, D), jnp.float32),
               jax.ShapeDtypeStruct((N, 16), jnp.float32)],
)

_tc2 = pl.pallas_call(
    _tc2_body,
    grid=(N // BN,),
    in_specs=[_part_spec(D), _row_spec(D), _row_spec(16), _full_spec(1, D),
              _full_spec(D, D)],
    out_specs=_row_spec(D),
    out_shape=jax.ShapeDtypeStruct((N, D), jnp.float32),
)

_tc3 = pl.pallas_call(
    _tc3_body,
    grid=(N // BN,),
    in_specs=[_part_spec(D), _row_spec(D), _row_spec(16), _full_spec(1, D)],
    out_specs=_row_spec(D),
    out_shape=jax.ShapeDtypeStruct((N, D), jnp.float32),
)


def kernel(edge_index, emb, W1, b1, W2, b2):
    ei = edge_index.astype(jnp.int32)
    pad = E_PAD - E
    rowr = jnp.concatenate([ei[0], jnp.zeros((pad,), jnp.int32)]
                           ).reshape(NW, NCH, C)
    colr = jnp.concatenate([ei[1], jnp.full((pad,), N, jnp.int32)]
                           ).reshape(NW, NCH, C)
    onesD = jnp.ones((C, D), jnp.float32)
    zD = jnp.zeros((NPAD, D), jnp.float32)

    cnt = _deg_call(colr, onesD, zD)
    g1, dis = _tc1(cnt, emb, W1)
    p = _agg_call(rowr, colr, g1, zD)
    g2 = _tc2(p, g1, dis, b1.reshape(1, D), W2)
    q = _agg_call(rowr, colr, g2, zD)
    return _tc3(q, g2, dis, b2.reshape(1, D))


# speedup vs baseline: 8.1610x; 1.1720x over previous
import functools

import jax
import jax.numpy as jnp
from jax import lax
from jax.experimental import pallas as pl
from jax.experimental.pallas import tpu as pltpu
from jax.experimental.pallas import tpu_sc as plsc

N = 10000
E = 320000
D = 128
NC = 2
NS = 16
NW = NC * NS
C = 128
NCH = 80
E_PAD = NW * NCH * C
NPAD = 10240
IPT = NPAD // NS
RPA = 624
TAIL0 = RPA * NS
TAILN = N - TAIL0

_mesh = plsc.VectorSubcoreMesh(core_axis_name="c", subcore_axis_name="s")


def _init_sliced(src, dst, s):
    pltpu.sync_copy(src.at[pl.ds(s * IPT, IPT)], dst.at[pl.ds(s * IPT, IPT)])


def _drain_sliced(src, dst, s):
    base = s * RPA
    pltpu.sync_copy(src.at[pl.ds(base, RPA)], dst.at[pl.ds(base, RPA)])

    @pl.when(s == NS - 1)
    def _():
        pltpu.sync_copy(src.at[pl.ds(TAIL0, TAILN)],
                        dst.at[pl.ds(TAIL0, TAILN)])


NP = NCH // 2


def _deg_body(colr, ones_h, zero_h, cnt_out, acc, coljA, coljB, onesb,
              semA, semB):
    c = lax.axis_index("c")
    s = lax.axis_index("s")
    wid = s * NC + c
    _init_sliced(zero_h, acc, s)
    pltpu.sync_copy(ones_h, onesb)
    pltpu.sync_copy(colr.at[wid, 0], coljA)
    plsc.subcore_barrier()

    def body(jj, carry):
        j1 = 2 * jj + 1
        j2 = 2 * jj + 2
        pltpu.async_copy(colr.at[wid, j1], coljB, semB)
        pltpu.sync_copy(onesb, acc.at[coljA], add=True)

        @pl.when(jj + 1 < NP)
        def _():
            pltpu.async_copy(colr.at[wid, j2], coljA, semA)

        pltpu.make_async_copy(colr.at[wid, j1], coljB, semB).wait()
        pltpu.sync_copy(onesb, acc.at[coljB], add=True)

        @pl.when(jj + 1 < NP)
        def _():
            pltpu.make_async_copy(colr.at[wid, j2], coljA, semA).wait()

        return carry

    lax.fori_loop(0, NP, body, 0)
    plsc.subcore_barrier()
    _drain_sliced(acc, cnt_out.at[c], s)


_deg_call = pl.kernel(
    _deg_body,
    out_type=jax.ShapeDtypeStruct((NC, N, D), jnp.float32),
    mesh=_mesh,
    scratch_types=[
        pltpu.VMEM_SHARED((NPAD, D), jnp.float32),
        pltpu.VMEM((C,), jnp.int32),
        pltpu.VMEM((C,), jnp.int32),
        pltpu.VMEM((C, D), jnp.float32),
        pltpu.SemaphoreType.DMA,
        pltpu.SemaphoreType.DMA,
    ],
)


def _agg_body(rowr, colr, g_h, zero_h, part_out, acc,
              rowjA, coljA, rowjB, coljB, rowsA, rowsB,
              semGA, semGB, semIA, semIB):
    c = lax.axis_index("c")
    s = lax.axis_index("s")
    wid = s * NC + c
    _init_sliced(zero_h, acc, s)
    pltpu.sync_copy(rowr.at[wid, 0], rowjA)
    pltpu.sync_copy(colr.at[wid, 0], coljA)
    pltpu.sync_copy(rowr.at[wid, 1], rowjB)
    pltpu.sync_copy(colr.at[wid, 1], coljB)
    plsc.subcore_barrier()
    pltpu.async_copy(g_h.at[rowjA], rowsA, semGA)

    def body(jj, carry):
        j2 = 2 * jj + 2
        j3 = 2 * jj + 3
        pltpu.async_copy(g_h.at[rowjB], rowsB, semGB)
        pltpu.make_async_copy(g_h.at[rowjA], rowsA, semGA).wait()
        pltpu.sync_copy(rowsA, acc.at[coljA], add=True)

        @pl.when(jj + 1 < NP)
        def _():
            pltpu.async_copy(rowr.at[wid, j2], rowjA, semIA)
            pltpu.async_copy(colr.at[wid, j2], coljA, semIA)

        pltpu.make_async_copy(g_h.at[rowjB], rowsB, semGB).wait()
        pltpu.sync_copy(rowsB, acc.at[coljB], add=True)

        @pl.when(jj + 1 < NP)
        def _():
            pltpu.async_copy(rowr.at[wid, j3], rowjB, semIB)
            pltpu.async_copy(colr.at[wid, j3], coljB, semIB)
            pltpu.make_async_copy(rowr.at[wid, j2], rowjA, semIA).wait()
            pltpu.make_async_copy(colr.at[wid, j2], coljA, semIA).wait()
            pltpu.async_copy(g_h.at[rowjA], rowsA, semGA)
            pltpu.make_async_copy(rowr.at[wid, j3], rowjB, semIB).wait()
            pltpu.make_async_copy(colr.at[wid, j3], coljB, semIB).wait()

        return carry

    lax.fori_loop(0, NP, body, 0)
    plsc.subcore_barrier()
    _drain_sliced(acc, part_out.at[c], s)


_agg_call = pl.kernel(
    _agg_body,
    out_type=jax.ShapeDtypeStruct((NC, N, D), jnp.float32),
    mesh=_mesh,
    scratch_types=[
        pltpu.VMEM_SHARED((NPAD, D), jnp.float32),
        pltpu.VMEM((C,), jnp.int32),
        pltpu.VMEM((C,), jnp.int32),
        pltpu.VMEM((C,), jnp.int32),
        pltpu.VMEM((C,), jnp.int32),
        pltpu.VMEM((C, D), jnp.float32),
        pltpu.VMEM((C, D), jnp.float32),
        pltpu.SemaphoreType.DMA,
        pltpu.SemaphoreType.DMA,
        pltpu.SemaphoreType.DMA,
        pltpu.SemaphoreType.DMA,
    ],
)

BN = 1000


def _tc1_body(cnt_ref, x_ref, w_ref, g_ref, dis_ref):
    deg = cnt_ref[0] + cnt_ref[1] + 1.0
    dis = lax.rsqrt(deg)
    dis_ref[...] = dis[:, 0:16]
    h = jnp.dot(x_ref[...], w_ref[...], preferred_element_type=jnp.float32)
    g_ref[...] = h * dis[:, 0:1]


def _tc2_body(p_ref, g1_ref, dis_ref, b1_ref, w2_ref, g2_ref):
    dis = dis_ref[...][:, 0:1]
    x1 = jnp.maximum(dis * (p_ref[0] + p_ref[1] + g1_ref[...]) + b1_ref[...],
                     0.0)
    h = jnp.dot(x1, w2_ref[...], preferred_element_type=jnp.float32)
    g2_ref[...] = h * dis


def _tc3_body(q_ref, g2_ref, dis_ref, b2_ref, o_ref):
    dis = dis_ref[...][:, 0:1]
    o_ref[...] = dis * (q_ref[0] + q_ref[1] + g2_ref[...]) + b2_ref[...]


def _row_spec(w):
    return pl.BlockSpec((BN, w), lambda i: (i, 0))


def _part_spec(w):
    return pl.BlockSpec((NC, BN, w), lambda i: (0, i, 0))


def _full_spec(r, c):
    return pl.BlockSpec((r, c), lambda i: (0, 0))


_tc1 = pl.pallas_call(
    _tc1_body,
    grid=(N // BN,),
    in_specs=[_part_spec(D), _row_spec(D), _full_spec(D, D)],
    out_specs=[_row_spec(D), _row_spec(16)],
    out_shape=[jax.ShapeDtypeStruct((N, D), jnp.float32),
               jax.ShapeDtypeStruct((N, 16), jnp.float32)],
)

_tc2 = pl.pallas_call(
    _tc2_body,
    grid=(N // BN,),
    in_specs=[_part_spec(D), _row_spec(D), _row_spec(16), _full_spec(1, D),
              _full_spec(D, D)],
    out_specs=_row_spec(D),
    out_shape=jax.ShapeDtypeStruct((N, D), jnp.float32),
)

_tc3 = pl.pallas_call(
    _tc3_body,
    grid=(N // BN,),
    in_specs=[_part_spec(D), _row_spec(D), _row_spec(16), _full_spec(1, D)],
    out_specs=_row_spec(D),
    out_shape=jax.ShapeDtypeStruct((N, D), jnp.float32),
)


def kernel(edge_index, emb, W1, b1, W2, b2):
    ei = edge_index.astype(jnp.int32)
    pad = E_PAD - E
    rowr = jnp.concatenate([ei[0], jnp.zeros((pad,), jnp.int32)]
                           ).reshape(NW, NCH, C)
    colr = jnp.concatenate([ei[1], jnp.full((pad,), N, jnp.int32)]
                           ).reshape(NW, NCH, C)
    onesD = jnp.ones((C, D), jnp.float32)
    zD = jnp.zeros((NPAD, D), jnp.float32)

    cnt = _deg_call(colr, onesD, zD)
    g1, dis = _tc1(cnt, emb, W1)
    p = _agg_call(rowr, colr, g1, zD)
    g2 = _tc2(p, g1, dis, b1.reshape(1, D), W2)
    q = _agg_call(rowr, colr, g2, zD)
    return _tc3(q, g2, dis, b2.reshape(1, D))


# speedup vs baseline: 9.2737x; 1.1364x over previous
import functools

import jax
import jax.numpy as jnp
from jax import lax
from jax.experimental import pallas as pl
from jax.experimental.pallas import tpu as pltpu
from jax.experimental.pallas import tpu_sc as plsc

N = 10000
E = 320000
D = 128
NC = 2
NS = 16
NW = NC * NS
C = 64
NCH = 160
E_PAD = NW * NCH * C
NPAD = 10240
IPT = NPAD // NS
RPA = 624
TAIL0 = RPA * NS
TAILN = N - TAIL0

_mesh = plsc.VectorSubcoreMesh(core_axis_name="c", subcore_axis_name="s")


def _init_sliced(src, dst, s):
    pltpu.sync_copy(src.at[pl.ds(s * IPT, IPT)], dst.at[pl.ds(s * IPT, IPT)])


def _drain_sliced(src, dst, s):
    base = s * RPA
    pltpu.sync_copy(src.at[pl.ds(base, RPA)], dst.at[pl.ds(base, RPA)])

    @pl.when(s == NS - 1)
    def _():
        pltpu.sync_copy(src.at[pl.ds(TAIL0, TAILN)],
                        dst.at[pl.ds(TAIL0, TAILN)])


NP = NCH // 2


def _deg_body(colr, ones_h, zero_h, cnt_out, acc, coljA, coljB, onesb,
              semA, semB):
    c = lax.axis_index("c")
    s = lax.axis_index("s")
    wid = s * NC + c
    _init_sliced(zero_h, acc, s)
    pltpu.sync_copy(ones_h, onesb)
    pltpu.sync_copy(colr.at[wid, 0], coljA)
    plsc.subcore_barrier()

    def body(jj, carry):
        j1 = 2 * jj + 1
        j2 = 2 * jj + 2
        pltpu.async_copy(colr.at[wid, j1], coljB, semB)
        pltpu.sync_copy(onesb, acc.at[coljA], add=True)

        @pl.when(jj + 1 < NP)
        def _():
            pltpu.async_copy(colr.at[wid, j2], coljA, semA)

        pltpu.make_async_copy(colr.at[wid, j1], coljB, semB).wait()
        pltpu.sync_copy(onesb, acc.at[coljB], add=True)

        @pl.when(jj + 1 < NP)
        def _():
            pltpu.make_async_copy(colr.at[wid, j2], coljA, semA).wait()

        return carry

    lax.fori_loop(0, NP, body, 0)
    plsc.subcore_barrier()
    _drain_sliced(acc, cnt_out.at[c], s)


_deg_call = pl.kernel(
    _deg_body,
    out_type=jax.ShapeDtypeStruct((NC, N, D), jnp.float32),
    mesh=_mesh,
    scratch_types=[
        pltpu.VMEM_SHARED((NPAD, D), jnp.float32),
        pltpu.VMEM((C,), jnp.int32),
        pltpu.VMEM((C,), jnp.int32),
        pltpu.VMEM((C, D), jnp.float32),
        pltpu.SemaphoreType.DMA,
        pltpu.SemaphoreType.DMA,
    ],
)


NB = 4


def _agg_body(rowr, colr, g_h, zero_h, part_out, *scr):
    acc = scr[0]
    rows = scr[1:1 + NB]
    rowj = (scr[5:9], scr[9:13])
    colj = (scr[13:17], scr[17:21])
    semG = scr[21:25]
    semI = (scr[25:29], scr[29:33])
    c = lax.axis_index("c")
    s = lax.axis_index("s")
    wid = s * NC + c
    _init_sliced(zero_h, acc, s)
    for b in range(NB):
        pltpu.sync_copy(rowr.at[wid, b], rowj[0][b])
        pltpu.sync_copy(colr.at[wid, b], colj[0][b])
    for b in range(NB):
        pltpu.async_copy(rowr.at[wid, NB + b], rowj[1][b], semI[1][b])
        pltpu.async_copy(colr.at[wid, NB + b], colj[1][b], semI[1][b])
    plsc.subcore_barrier()
    for b in range(NB):
        pltpu.async_copy(g_h.at[rowj[0][b]], rows[b], semG[b])

    def body(jj, carry):
        q0 = 2 * NB * jj
        for k in range(2 * NB):
            q = q0 + k
            p, b = (k // NB) % 2, k % NB
            pn = 1 - p
            pltpu.make_async_copy(g_h.at[rowj[p][b]], rows[b],
                                  semG[b]).wait()
            pltpu.sync_copy(rows[b], acc.at[colj[p][b]], add=True)

            @pl.when(q + 2 * NB < NCH)
            def _():
                pltpu.async_copy(rowr.at[wid, q + 2 * NB], rowj[p][b],
                                 semI[p][b])
                pltpu.async_copy(colr.at[wid, q + 2 * NB], colj[p][b],
                                 semI[p][b])

            @pl.when(q + NB < NCH)
            def _():
                pltpu.make_async_copy(rowr.at[wid, q + NB], rowj[pn][b],
                                      semI[pn][b]).wait()
                pltpu.make_async_copy(colr.at[wid, q + NB], colj[pn][b],
                                      semI[pn][b]).wait()
                pltpu.async_copy(g_h.at[rowj[pn][b]], rows[b], semG[b])

        return carry

    lax.fori_loop(0, NCH // (2 * NB), body, 0)
    plsc.subcore_barrier()
    _drain_sliced(acc, part_out.at[c], s)


_agg_call = pl.kernel(
    _agg_body,
    out_type=jax.ShapeDtypeStruct((NC, N, D), jnp.float32),
    mesh=_mesh,
    scratch_types=(
        [pltpu.VMEM_SHARED((NPAD, D), jnp.float32)]
        + [pltpu.VMEM((C, D), jnp.float32)] * NB
        + [pltpu.VMEM((C,), jnp.int32)] * (4 * NB)
        + [pltpu.SemaphoreType.DMA] * (3 * NB)
    ),
)

BN = 1000


def _tc1_body(cnt_ref, x_ref, w_ref, g_ref, dis_ref):
    deg = cnt_ref[0] + cnt_ref[1] + 1.0
    dis = lax.rsqrt(deg)
    dis_ref[...] = dis[:, 0:16]
    h = jnp.dot(x_ref[...], w_ref[...], preferred_element_type=jnp.float32)
    g_ref[...] = h * dis[:, 0:1]


def _tc2_body(p_ref, g1_ref, dis_ref, b1_ref, w2_ref, g2_ref):
    dis = dis_ref[...][:, 0:1]
    x1 = jnp.maximum(dis * (p_ref[0] + p_ref[1] + g1_ref[...]) + b1_ref[...],
                     0.0)
    h = jnp.dot(x1, w2_ref[...], preferred_element_type=jnp.float32)
    g2_ref[...] = h * dis


def _tc3_body(q_ref, g2_ref, dis_ref, b2_ref, o_ref):
    dis = dis_ref[...][:, 0:1]
    o_ref[...] = dis * (q_ref[0] + q_ref[1] + g2_ref[...]) + b2_ref[...]


def _row_spec(w):
    return pl.BlockSpec((BN, w), lambda i: (i, 0))


def _part_spec(w):
    return pl.BlockSpec((NC, BN, w), lambda i: (0, i, 0))


def _full_spec(r, c):
    return pl.BlockSpec((r, c), lambda i: (0, 0))


_tc1 = pl.pallas_call(
    _tc1_body,
    grid=(N // BN,),
    in_specs=[_part_spec(D), _row_spec(D), _full_spec(D, D)],
    out_specs=[_row_spec(D), _row_spec(16)],
    out_shape=[jax.ShapeDtypeStruct((N, D), jnp.float32),
               jax.ShapeDtypeStruct((N, 16), jnp.float32)],
)

_tc2 = pl.pallas_call(
    _tc2_body,
    grid=(N // BN,),
    in_specs=[_part_spec(D), _row_spec(D), _row_spec(16), _full_spec(1, D),
              _full_spec(D, D)],
    out_specs=_row_spec(D),
    out_shape=jax.ShapeDtypeStruct((N, D), jnp.float32),
)

_tc3 = pl.pallas_call(
    _tc3_body,
    grid=(N // BN,),
    in_specs=[_part_spec(D), _row_spec(D), _row_spec(16), _full_spec(1, D)],
    out_specs=_row_spec(D),
    out_shape=jax.ShapeDtypeStruct((N, D), jnp.float32),
)


def kernel(edge_index, emb, W1, b1, W2, b2):
    ei = edge_index.astype(jnp.int32)
    pad = E_PAD - E
    rowr = jnp.concatenate([ei[0], jnp.zeros((pad,), jnp.int32)]
                           ).reshape(NW, NCH, C)
    colr = jnp.concatenate([ei[1], jnp.full((pad,), N, jnp.int32)]
                           ).reshape(NW, NCH, C)
    onesD = jnp.ones((C, D), jnp.float32)
    zD = jnp.zeros((NPAD, D), jnp.float32)

    cnt = _deg_call(colr, onesD, zD)
    g1, dis = _tc1(cnt, emb, W1)
    p = _agg_call(rowr, colr, g1, zD)
    g2 = _tc2(p, g1, dis, b1.reshape(1, D), W2)
    q = _agg_call(rowr, colr, g2, zD)
    return _tc3(q, g2, dis, b2.reshape(1, D))


# speedup vs baseline: 10.5300x; 1.1355x over previous
import functools

import jax
import jax.numpy as jnp
from jax import lax
from jax.experimental import pallas as pl
from jax.experimental.pallas import tpu as pltpu
from jax.experimental.pallas import tpu_sc as plsc

N = 10000
E = 320000
D = 128
NC = 2
NS = 16
NW = NC * NS
C = 64
NCH = 160
NCH2 = 2 * NCH
K0 = 264
K1 = NCH2 - K0
E_PAD = NW * NCH * C
NPAD = 10240
IPT = NPAD // NS
RPA = 624
TAIL0 = RPA * NS
TAILN = N - TAIL0

_mesh = plsc.VectorSubcoreMesh(core_axis_name="c", subcore_axis_name="s")


def _init_sliced(src, dst, s):
    pltpu.sync_copy(src.at[pl.ds(s * IPT, IPT)], dst.at[pl.ds(s * IPT, IPT)])


def _drain_sliced(src, dst, s):
    base = s * RPA
    pltpu.sync_copy(src.at[pl.ds(base, RPA)], dst.at[pl.ds(base, RPA)])

    @pl.when(s == NS - 1)
    def _():
        pltpu.sync_copy(src.at[pl.ds(TAIL0, TAILN)],
                        dst.at[pl.ds(TAIL0, TAILN)])


NP = NCH // 2


def _deg_body(colr, ones_h, zero_h, cnt_out, acc, coljA, coljB, onesb,
              semA, semB):
    c = lax.axis_index("c")
    s = lax.axis_index("s")
    cb = c * NCH
    _init_sliced(zero_h, acc, s)
    pltpu.sync_copy(ones_h, onesb)
    pltpu.sync_copy(colr.at[s, cb], coljA)
    plsc.subcore_barrier()

    def body(jj, carry):
        j1 = cb + 2 * jj + 1
        j2 = cb + 2 * jj + 2
        pltpu.async_copy(colr.at[s, j1], coljB, semB)
        pltpu.sync_copy(onesb, acc.at[coljA], add=True)

        @pl.when(jj + 1 < NP)
        def _():
            pltpu.async_copy(colr.at[s, j2], coljA, semA)

        pltpu.make_async_copy(colr.at[s, j1], coljB, semB).wait()
        pltpu.sync_copy(onesb, acc.at[coljB], add=True)

        @pl.when(jj + 1 < NP)
        def _():
            pltpu.make_async_copy(colr.at[s, j2], coljA, semA).wait()

        return carry

    lax.fori_loop(0, NP, body, 0)
    plsc.subcore_barrier()
    _drain_sliced(acc, cnt_out.at[c], s)


_deg_call = pl.kernel(
    _deg_body,
    out_type=jax.ShapeDtypeStruct((NC, N, D), jnp.float32),
    mesh=_mesh,
    scratch_types=[
        pltpu.VMEM_SHARED((NPAD, D), jnp.float32),
        pltpu.VMEM((C,), jnp.int32),
        pltpu.VMEM((C,), jnp.int32),
        pltpu.VMEM((C, D), jnp.float32),
        pltpu.SemaphoreType.DMA,
        pltpu.SemaphoreType.DMA,
    ],
)


NB = 4


def _agg_body(rowr, colr, g_h, zero_h, part_out, *scr):
    acc = scr[0]
    rows = scr[1:1 + NB]
    rowj = (scr[5:9], scr[9:13])
    colj = (scr[13:17], scr[17:21])
    semG = scr[21:25]
    semI = (scr[25:29], scr[29:33])
    c = lax.axis_index("c")
    s = lax.axis_index("s")
    cb = jnp.where(c == 0, 0, K0)
    nch_c = jnp.where(c == 0, K0, K1)
    _init_sliced(zero_h, acc, s)
    for b in range(NB):
        pltpu.sync_copy(rowr.at[s, cb + b], rowj[0][b])
        pltpu.sync_copy(colr.at[s, cb + b], colj[0][b])
    for b in range(NB):
        pltpu.async_copy(rowr.at[s, cb + NB + b], rowj[1][b], semI[1][b])
        pltpu.async_copy(colr.at[s, cb + NB + b], colj[1][b], semI[1][b])
    plsc.subcore_barrier()
    for b in range(NB):
        pltpu.async_copy(g_h.at[rowj[0][b]], rows[b], semG[b])

    def body(jj, carry):
        q0 = 2 * NB * jj
        for k in range(2 * NB):
            q = q0 + k
            p, b = (k // NB) % 2, k % NB
            pn = 1 - p
            pltpu.make_async_copy(g_h.at[rowj[p][b]], rows[b],
                                  semG[b]).wait()
            pltpu.sync_copy(rows[b], acc.at[colj[p][b]], add=True)

            @pl.when(q + 2 * NB < nch_c)
            def _():
                pltpu.async_copy(rowr.at[s, cb + q + 2 * NB], rowj[p][b],
                                 semI[p][b])
                pltpu.async_copy(colr.at[s, cb + q + 2 * NB], colj[p][b],
                                 semI[p][b])

            @pl.when(q + NB < nch_c)
            def _():
                pltpu.make_async_copy(rowr.at[s, cb + q + NB], rowj[pn][b],
                                      semI[pn][b]).wait()
                pltpu.make_async_copy(colr.at[s, cb + q + NB], colj[pn][b],
                                      semI[pn][b]).wait()
                pltpu.async_copy(g_h.at[rowj[pn][b]], rows[b], semG[b])

        return carry

    lax.fori_loop(0, nch_c // (2 * NB), body, 0)
    plsc.subcore_barrier()
    _drain_sliced(acc, part_out.at[c], s)


_agg_call = pl.kernel(
    _agg_body,
    out_type=jax.ShapeDtypeStruct((NC, N, D), jnp.float32),
    mesh=_mesh,
    scratch_types=(
        [pltpu.VMEM_SHARED((NPAD, D), jnp.float32)]
        + [pltpu.VMEM((C, D), jnp.float32)] * NB
        + [pltpu.VMEM((C,), jnp.int32)] * (4 * NB)
        + [pltpu.SemaphoreType.DMA] * (3 * NB)
    ),
)

BN = 1000


def _tc1_body(cnt_ref, x_ref, w_ref, g_ref, dis_ref):
    deg = cnt_ref[0] + cnt_ref[1] + 1.0
    dis = lax.rsqrt(deg)
    dis_ref[...] = dis[:, 0:16]
    h = jnp.dot(x_ref[...], w_ref[...], preferred_element_type=jnp.float32)
    g_ref[...] = h * dis[:, 0:1]


def _tc2_body(p_ref, g1_ref, dis_ref, b1_ref, w2_ref, g2_ref):
    dis = dis_ref[...][:, 0:1]
    x1 = jnp.maximum(dis * (p_ref[0] + p_ref[1] + g1_ref[...]) + b1_ref[...],
                     0.0)
    h = jnp.dot(x1, w2_ref[...], preferred_element_type=jnp.float32)
    g2_ref[...] = h * dis


def _tc3_body(q_ref, g2_ref, dis_ref, b2_ref, o_ref):
    dis = dis_ref[...][:, 0:1]
    o_ref[...] = dis * (q_ref[0] + q_ref[1] + g2_ref[...]) + b2_ref[...]


def _row_spec(w):
    return pl.BlockSpec((BN, w), lambda i: (i, 0))


def _part_spec(w):
    return pl.BlockSpec((NC, BN, w), lambda i: (0, i, 0))


def _full_spec(r, c):
    return pl.BlockSpec((r, c), lambda i: (0, 0))


_tc1 = pl.pallas_call(
    _tc1_body,
    grid=(N // BN,),
    in_specs=[_part_spec(D), _row_spec(D), _full_spec(D, D)],
    out_specs=[_row_spec(D), _row_spec(16)],
    out_shape=[jax.ShapeDtypeStruct((N, D), jnp.float32),
               jax.ShapeDtypeStruct((N, 16), jnp.float32)],
)

_tc2 = pl.pallas_call(
    _tc2_body,
    grid=(N // BN,),
    in_specs=[_part_spec(D), _row_spec(D), _row_spec(16), _full_spec(1, D),
              _full_spec(D, D)],
    out_specs=_row_spec(D),
    out_shape=jax.ShapeDtypeStruct((N, D), jnp.float32),
)

_tc3 = pl.pallas_call(
    _tc3_body,
    grid=(N // BN,),
    in_specs=[_part_spec(D), _row_spec(D), _row_spec(16), _full_spec(1, D)],
    out_specs=_row_spec(D),
    out_shape=jax.ShapeDtypeStruct((N, D), jnp.float32),
)


def kernel(edge_index, emb, W1, b1, W2, b2):
    ei = edge_index.astype(jnp.int32)
    pad = E_PAD - E
    rowr = jnp.concatenate([ei[0], jnp.zeros((pad,), jnp.int32)]
                           ).reshape(NS, NCH2, C)
    colr = jnp.concatenate([ei[1], jnp.full((pad,), N, jnp.int32)]
                           ).reshape(NS, NCH2, C)
    onesD = jnp.ones((C, D), jnp.float32)
    zD = jnp.zeros((NPAD, D), jnp.float32)

    cnt = _deg_call(colr, onesD, zD)
    g1, dis = _tc1(cnt, emb, W1)
    p = _agg_call(rowr, colr, g1, zD)
    g2 = _tc2(p, g1, dis, b1.reshape(1, D), W2)
    q = _agg_call(rowr, colr, g2, zD)
    return _tc3(q, g2, dis, b2.reshape(1, D))
